# baseline (speedup 1.0000x reference)
import functools

import jax
import jax.numpy as jnp
from jax import lax
from jax.experimental import pallas as pl
from jax.experimental.pallas import tpu as pltpu
from jax.experimental.pallas import tpu_sc as plsc

NC, NS = 2, 16
NW = NC * NS
BATCH, SEQ, EMBED = 4096, 200, 64
BPW = BATCH // NW
CH = 104
NCH = 2


def _sc_body(idx_hbm, table_hbm, w_hbm, b_hbm, out_hbm,
             idx_v, rows0, rows1, w_v, b_v, out_v,
             sem0, sem1):
    wid = lax.axis_index("s") * NC + lax.axis_index("c")
    base = wid * BPW
    pltpu.sync_copy(idx_hbm.at[pl.ds(base, BPW)], idx_v)
    pltpu.sync_copy(w_hbm, w_v)
    pltpu.sync_copy(b_hbm, b_v)
    w0 = w_v[pl.ds(0, 16)]
    w1 = w_v[pl.ds(16, 16)]
    w2 = w_v[pl.ds(32, 16)]
    w3 = w_v[pl.ds(48, 16)]
    bias = b_v[pl.ds(0, 16)][0]
    lane0 = lax.iota(jnp.int32, 16) == 0

    def batch_body(b, carry):
        cp0 = pltpu.async_copy(table_hbm.at[idx_v.at[b, 0]], rows0, sem0)
        cp1 = pltpu.async_copy(table_hbm.at[idx_v.at[b, 1]], rows1, sem1)
        cp0.wait()
        cp1.wait()

        def red0(r, acc):
            a0, a1, a2, a3 = acc
            return (jnp.maximum(a0, rows0[r, pl.ds(0, 16)]),
                    jnp.maximum(a1, rows0[r, pl.ds(16, 16)]),
                    jnp.maximum(a2, rows0[r, pl.ds(32, 16)]),
                    jnp.maximum(a3, rows0[r, pl.ds(48, 16)]))

        def red1(r, acc):
            a0, a1, a2, a3 = acc
            return (jnp.maximum(a0, rows1[r, pl.ds(0, 16)]),
                    jnp.maximum(a1, rows1[r, pl.ds(16, 16)]),
                    jnp.maximum(a2, rows1[r, pl.ds(32, 16)]),
                    jnp.maximum(a3, rows1[r, pl.ds(48, 16)]))

        acc0 = (rows0[0, pl.ds(0, 16)], rows0[0, pl.ds(16, 16)],
                rows0[0, pl.ds(32, 16)], rows0[0, pl.ds(48, 16)])
        acc1 = lax.fori_loop(1, CH, red0, acc0)
        a0, a1, a2, a3 = lax.fori_loop(0, CH, red1, acc1)
        t = a0 * w0 + a1 * w1 + a2 * w2 + a3 * w3
        s = jnp.sum(t) + bias
        plsc.store_scatter(out_v, [jnp.full((16,), b, jnp.int32)],
                           jnp.broadcast_to(s, (16,)), mask=lane0)
        return carry

    lax.fori_loop(0, BPW, batch_body, 0)
    pltpu.sync_copy(out_v, out_hbm.at[pl.ds(base, BPW)])


@jax.jit
def _sc_call(idx, table, w, b):
    mesh = plsc.VectorSubcoreMesh(core_axis_name="c", subcore_axis_name="s")
    return pl.kernel(
        _sc_body,
        out_type=jax.ShapeDtypeStruct((BATCH,), jnp.float32),
        mesh=mesh,
        scratch_types=[
            pltpu.VMEM((BPW, NCH, CH), jnp.int32),
            pltpu.VMEM((CH, EMBED), jnp.float32),
            pltpu.VMEM((CH, EMBED), jnp.float32),
            pltpu.VMEM((EMBED,), jnp.float32),
            pltpu.VMEM((16,), jnp.float32),
            pltpu.VMEM((BPW,), jnp.float32),
            pltpu.SemaphoreType.DMA,
            pltpu.SemaphoreType.DMA,
        ],
        compiler_params=pltpu.CompilerParams(
            use_tc_tiling_on_sc=False, needs_layout_passes=False),
    )(idx, table, w, b)


def kernel(context, endings, embed_table, fc_w, fc_b):
    idx = endings[0].T
    idx = jnp.concatenate([idx, idx[:, :NCH * CH - SEQ]], axis=1)
    idx = idx.reshape(BATCH, NCH, CH)
    w = fc_w.reshape(EMBED)
    b = jnp.broadcast_to(fc_b, (16,))
    return _sc_call(idx, embed_table, w, b)


# speedup vs baseline: 1.1692x; 1.1692x over previous
import functools

import jax
import jax.numpy as jnp
from jax import lax
from jax.experimental import pallas as pl
from jax.experimental.pallas import tpu as pltpu
from jax.experimental.pallas import tpu_sc as plsc

NC, NS = 2, 16
NW = NC * NS
BATCH, SEQ, EMBED = 4096, 200, 64
BPW = BATCH // NW
CH = 104
NCH = 2


RU = 8


def _sc_body(idx_hbm, table_hbm, w_hbm, b_hbm, out_hbm,
             idx_v, rows00, rows01, rows10, rows11, w_v, b_v, out_v,
             sem00, sem01, sem10, sem11):
    wid = lax.axis_index("s") * NC + lax.axis_index("c")
    base = wid * BPW
    pltpu.sync_copy(idx_hbm.at[pl.ds(base, BPW)], idx_v)
    pltpu.sync_copy(w_hbm, w_v)
    pltpu.sync_copy(b_hbm, b_v)
    w0 = w_v[pl.ds(0, 16)]
    w1 = w_v[pl.ds(16, 16)]
    w2 = w_v[pl.ds(32, 16)]
    w3 = w_v[pl.ds(48, 16)]
    bias = b_v[pl.ds(0, 16)][0]
    lane0 = lax.iota(jnp.int32, 16) == 0
    ninf = jnp.full((16,), -jnp.inf, jnp.float32)

    bufs = ((rows00, rows01, sem00, sem01),
            (rows10, rows11, sem10, sem11))

    def issue(b, slot):
        r0, r1, s0, s1 = bufs[slot]
        pltpu.async_copy(table_hbm.at[idx_v.at[b, 0]], r0, s0)
        pltpu.async_copy(table_hbm.at[idx_v.at[b, 1]], r1, s1)

    def reduce_chunk(rref, acc):
        def body(i, acc):
            a0, a1, a2, a3 = acc
            for j in range(RU):
                r = i * RU + j
                a0 = jnp.maximum(a0, rref[r, pl.ds(0, 16)])
                a1 = jnp.maximum(a1, rref[r, pl.ds(16, 16)])
                a2 = jnp.maximum(a2, rref[r, pl.ds(32, 16)])
                a3 = jnp.maximum(a3, rref[r, pl.ds(48, 16)])
            return (a0, a1, a2, a3)
        return lax.fori_loop(0, CH // RU, body, acc)

    def consume(b, slot):
        r0, r1, s0, s1 = bufs[slot]
        pltpu.make_async_copy(table_hbm.at[idx_v.at[b, 0]], r0, s0).wait()
        pltpu.make_async_copy(table_hbm.at[idx_v.at[b, 1]], r1, s1).wait()
        acc = reduce_chunk(r0, (ninf, ninf, ninf, ninf))
        a0, a1, a2, a3 = reduce_chunk(r1, acc)
        t = a0 * w0 + a1 * w1 + a2 * w2 + a3 * w3
        s = jnp.sum(t) + bias
        plsc.store_scatter(out_v, [jnp.full((16,), b, jnp.int32)],
                           jnp.broadcast_to(s, (16,)), mask=lane0)

    issue(0, 0)
    issue(1, 1)

    def pipe(g, carry):
        b = 2 * g
        consume(b, 0)
        issue(b + 2, 0)
        consume(b + 1, 1)
        issue(b + 3, 1)
        return carry

    lax.fori_loop(0, BPW // 2 - 1, pipe, 0)
    consume(BPW - 2, 0)
    consume(BPW - 1, 1)
    pltpu.sync_copy(out_v, out_hbm.at[pl.ds(base, BPW)])


@jax.jit
def _sc_call(idx, table, w, b):
    mesh = plsc.VectorSubcoreMesh(core_axis_name="c", subcore_axis_name="s")
    return pl.kernel(
        _sc_body,
        out_type=jax.ShapeDtypeStruct((BATCH,), jnp.float32),
        mesh=mesh,
        scratch_types=[
            pltpu.VMEM((BPW, NCH, CH), jnp.int32),
            pltpu.VMEM((CH, EMBED), jnp.float32),
            pltpu.VMEM((CH, EMBED), jnp.float32),
            pltpu.VMEM((CH, EMBED), jnp.float32),
            pltpu.VMEM((CH, EMBED), jnp.float32),
            pltpu.VMEM((EMBED,), jnp.float32),
            pltpu.VMEM((16,), jnp.float32),
            pltpu.VMEM((BPW,), jnp.float32),
            pltpu.SemaphoreType.DMA,
            pltpu.SemaphoreType.DMA,
            pltpu.SemaphoreType.DMA,
            pltpu.SemaphoreType.DMA,
        ],
        compiler_params=pltpu.CompilerParams(
            use_tc_tiling_on_sc=False, needs_layout_passes=False),
    )(idx, table, w, b)


def kernel(context, endings, embed_table, fc_w, fc_b):
    idx = endings[0].T
    idx = jnp.concatenate([idx, idx[:, :NCH * CH - SEQ]], axis=1)
    idx = idx.reshape(BATCH, NCH, CH)
    w = fc_w.reshape(EMBED)
    b = jnp.broadcast_to(fc_b, (16,))
    return _sc_call(idx, embed_table, w, b)
